# Initial kernel scaffold; baseline (speedup 1.0000x reference)
import jax
import jax.numpy as jnp
from jax.experimental import pallas as pl


def kernel(token_ids, weight):
    raise NotImplementedError("write your pallas kernel here")


# speedup vs baseline: 1.2396x; 1.2396x over previous
import functools

import jax
import jax.numpy as jnp
from jax import lax
from jax.experimental import pallas as pl
from jax.experimental.pallas import tpu as pltpu
from jax.experimental.pallas import tpu_sc as plsc

D = 32
B_ROWS = 16384 * 26
NC = 2
NS = 16
NW = NC * NS
CHUNK = 128
ROWS_PER_W = B_ROWS // NW
NCHUNKS = ROWS_PER_W // CHUNK

_mesh = plsc.VectorSubcoreMesh(core_axis_name="c", subcore_axis_name="s")


@functools.partial(
    pl.kernel,
    mesh=_mesh,
    out_type=jax.ShapeDtypeStruct((NW, NCHUNKS, CHUNK, D), jnp.float32),
    scratch_types=[
        pltpu.VMEM((NCHUNKS, CHUNK), jnp.int32),
        pltpu.VMEM((CHUNK, D), jnp.float32),
        pltpu.SemaphoreType.DMA,
    ],
    compiler_params=pltpu.CompilerParams(use_tc_tiling_on_sc=False),
)
def _gather_kernel(idx_hbm, table_hbm, out_hbm, idx_v, rows_v, sem):
    wid = lax.axis_index("s") * NC + lax.axis_index("c")
    pltpu.sync_copy(idx_hbm.at[wid], idx_v)

    def step(j, carry):
        pltpu.async_copy(table_hbm.at[idx_v.at[j]], rows_v, sem).wait()
        pltpu.sync_copy(rows_v, out_hbm.at[wid, j])
        return carry

    lax.fori_loop(0, NCHUNKS, step, 0)


def kernel(token_ids, weight):
    ids = token_ids.astype(jnp.int32).reshape(NW, NCHUNKS, CHUNK)
    out = _gather_kernel(ids, weight)
    return out.reshape(token_ids.shape[0], token_ids.shape[1], D)


# speedup vs baseline: 1.3563x; 1.0942x over previous
import functools

import jax
import jax.numpy as jnp
from jax import lax
from jax.experimental import pallas as pl
from jax.experimental.pallas import tpu as pltpu
from jax.experimental.pallas import tpu_sc as plsc

D = 32
B_ROWS = 16384 * 26
NC = 2
NS = 16
NW = NC * NS
CHUNK = 128
ROWS_PER_W = B_ROWS // NW
NCHUNKS = ROWS_PER_W // CHUNK
NBUF = 4
NG = NCHUNKS // NBUF

_mesh = plsc.VectorSubcoreMesh(core_axis_name="c", subcore_axis_name="s")


@functools.partial(
    pl.kernel,
    mesh=_mesh,
    out_type=jax.ShapeDtypeStruct((NW, NCHUNKS, CHUNK, D), jnp.float32),
    scratch_types=[
        pltpu.VMEM((NCHUNKS, CHUNK), jnp.int32),
        pltpu.VMEM((NBUF, CHUNK, D), jnp.float32),
        [pltpu.SemaphoreType.DMA] * NBUF,
        [pltpu.SemaphoreType.DMA] * NBUF,
    ],
    compiler_params=pltpu.CompilerParams(use_tc_tiling_on_sc=False),
)
def _gather_kernel(idx_hbm, table_hbm, out_hbm, idx_v, rows_v, sems_g, sems_w):
    wid = lax.axis_index("s") * NC + lax.axis_index("c")
    pltpu.sync_copy(idx_hbm.at[wid], idx_v)

    def gather(j, b):
        return pltpu.make_async_copy(table_hbm.at[idx_v.at[j]], rows_v.at[b],
                                     sems_g[b])

    def write(j, b):
        return pltpu.make_async_copy(rows_v.at[b], out_hbm.at[wid, j],
                                     sems_w[b])

    for b in range(NBUF):
        gather(b, b).start()

    def group(g, carry):
        for b in range(NBUF):
            j = g * NBUF + b
            gather(j, b).wait()
            w = write(j, b)
            w.start()
            w.wait()
            gather(j + NBUF, b).start()
        return carry

    lax.fori_loop(0, NG - 1, group, 0)

    for b in range(NBUF):
        j = (NG - 1) * NBUF + b
        gather(j, b).wait()
        w = write(j, b)
        w.start()
        w.wait()


def kernel(token_ids, weight):
    ids = token_ids.astype(jnp.int32).reshape(NW, NCHUNKS, CHUNK)
    out = _gather_kernel(ids, weight)
    return out.reshape(token_ids.shape[0], token_ids.shape[1], D)
